# baseline (speedup 1.0000x reference)
import functools

import jax
import jax.numpy as jnp
from jax import lax
from jax.experimental import pallas as pl
from jax.experimental.pallas import tpu as pltpu
from jax.experimental.pallas import tpu_sc as plsc

_N = 10000
_E = 320000
_D_IN = 128
_D_EDGE = 16
_NG = 64
_MEMBERS = [(-1.645, 0.283), (-1.08, 0.17), (-0.739, 0.134), (-0.468, 0.118),
            (-0.228, 0.114), (0.0, 0.114), (0.228, 0.114), (0.468, 0.118),
            (0.739, 0.134), (1.08, 0.17), (1.645, 0.283)]

_NC = 2
_NS = 16
_NW = _NC * _NS
_EPW = _E // _NW
_CK = 80
_NCH = _EPW // _CK
_NPAD = 10240
_HALF = _NPAD // 2
_CKM = 1000
_NCHM = _EPW // _CKM

_HP = lax.Precision.HIGHEST


def _dot(a, b):
    return jnp.dot(a, b, precision=_HP, preferred_element_type=jnp.float32)


def _node_body(hv_ref, wt_ref, bt_ref, waa_ref, baa_ref, t_ref, haa_ref):
    x = hv_ref[...]
    t_ref[...] = _dot(x, wt_ref[...]) + bt_ref[...]
    haa_ref[...] = jnp.maximum(_dot(x, waa_ref[...]) + baa_ref[...], 0.0)


def _node_call(hv, wt, bt, waa, baa):
    blk = 2000
    grid = _N // blk
    return pl.pallas_call(
        _node_body,
        grid=(grid,),
        in_specs=[
            pl.BlockSpec((blk, _D_IN), lambda i: (i, 0)),
            pl.BlockSpec((_D_IN, 2 * _D_IN), lambda i: (0, 0)),
            pl.BlockSpec((1, 2 * _D_IN), lambda i: (0, 0)),
            pl.BlockSpec((_D_IN, _D_IN), lambda i: (0, 0)),
            pl.BlockSpec((1, _D_IN), lambda i: (0, 0)),
        ],
        out_specs=[
            pl.BlockSpec((blk, 2 * _D_IN), lambda i: (i, 0)),
            pl.BlockSpec((blk, _D_IN), lambda i: (i, 0)),
        ],
        out_shape=[
            jax.ShapeDtypeStruct((_N, 2 * _D_IN), jnp.float32),
            jax.ShapeDtypeStruct((_N, _D_IN), jnp.float32),
        ],
    )(hv, wt, bt, waa, baa)


def _edgepre_body(he_ref, wc_ref, bc_ref, p2t_ref, pb_ref, msg_ref, r2_ref):
    t = jnp.maximum(_dot(he_ref[...], wc_ref[...]) + bc_ref[...], 0.0)
    msg_ref[...] = t[:, :_D_EDGE]
    r2_ref[...] = _dot(t[:, _D_EDGE:], p2t_ref[...]) + pb_ref[...]


def _edgepre_call(he, wc, bc, p2t, pb):
    blk = 8000
    grid = _E // blk
    return pl.pallas_call(
        _edgepre_body,
        grid=(grid,),
        in_specs=[
            pl.BlockSpec((blk, _D_EDGE), lambda i: (i, 0)),
            pl.BlockSpec((_D_EDGE, 2 * _D_EDGE), lambda i: (0, 0)),
            pl.BlockSpec((1, 2 * _D_EDGE), lambda i: (0, 0)),
            pl.BlockSpec((_D_EDGE, _D_EDGE), lambda i: (0, 0)),
            pl.BlockSpec((1, _D_EDGE), lambda i: (0, 0)),
        ],
        out_specs=[
            pl.BlockSpec((blk, _D_EDGE), lambda i: (i, 0)),
            pl.BlockSpec((blk, _D_EDGE), lambda i: (i, 0)),
        ],
        out_shape=[
            jax.ShapeDtypeStruct((_E, _D_EDGE), jnp.float32),
            jax.ShapeDtypeStruct((_E, _D_EDGE), jnp.float32),
        ],
    )(he, wc, bc, p2t, pb)


def _sc_q_body(t_hbm, src3_hbm, dst3_hbm, q_hbm,
               idx_s, idx_d, rows_s, rows_d, out_v,
               sem_s, sem_d):
    c = lax.axis_index("c")
    s = lax.axis_index("s")
    w = c * _NS + s
    base = w * _EPW

    pltpu.sync_copy(src3_hbm.at[w], idx_s)
    pltpu.sync_copy(dst3_hbm.at[w], idx_d)

    def chunk(j, carry):
        eb = base + j * _CK
        cp_s = pltpu.async_copy(t_hbm.at[idx_s.at[j]], rows_s, sem_s)
        cp_d = pltpu.async_copy(t_hbm.at[idx_d.at[j]], rows_d, sem_d)
        cp_s.wait()
        cp_d.wait()

        def edge(e, carry2):
            for cc in range(_D_IN // 16):
                lo = cc * 16
                a = rows_s[e, pl.ds(lo, 16)] + rows_d[e, pl.ds(_D_IN + lo, 16)]
                b = rows_d[e, pl.ds(lo, 16)] + rows_s[e, pl.ds(_D_IN + lo, 16)]
                out_v[e, pl.ds(lo, 16)] = (jnp.maximum(a, 0.0)
                                           + jnp.maximum(b, 0.0))
            return carry2

        lax.fori_loop(0, _CK, edge, 0)
        pltpu.sync_copy(out_v, q_hbm.at[pl.ds(eb, _CK)])
        return carry

    lax.fori_loop(0, _NCH, chunk, 0)


def _sc_q_call(t_tab, src3, dst3):
    mesh = plsc.VectorSubcoreMesh(core_axis_name="c", subcore_axis_name="s")
    f = functools.partial(
        pl.kernel, mesh=mesh,
        out_type=jax.ShapeDtypeStruct((_E, _D_IN), jnp.float32),
        scratch_types=[
            pltpu.VMEM((_NCH, _CK), jnp.int32),
            pltpu.VMEM((_NCH, _CK), jnp.int32),
            pltpu.VMEM((_CK, 2 * _D_IN), jnp.float32),
            pltpu.VMEM((_CK, 2 * _D_IN), jnp.float32),
            pltpu.VMEM((_CK, _D_IN), jnp.float32),
            pltpu.SemaphoreType.DMA,
            pltpu.SemaphoreType.DMA,
        ],
    )(_sc_q_body)
    return f(t_tab, src3, dst3)


def _sc_m_body(msg3_hbm, dst3_hbm, zer_hbm, mp_hbm,
               acc, msg_v, dst_v):
    c = lax.axis_index("c")
    s = lax.axis_index("s")
    w = c * _NS + s
    cidx = lax.iota(jnp.int32, 16)

    for p in range(2):
        base_flat = p * _HALF * _D_EDGE
        pltpu.sync_copy(zer_hbm, acc.at[pl.ds(0, _HALF * _D_EDGE)])

        def chunk(j, carry):
            pltpu.sync_copy(msg3_hbm.at[w, j], msg_v)
            pltpu.sync_copy(dst3_hbm.at[w, j], dst_v)

            def edge(e, carry2):
                row = msg_v[pl.ds(e * _D_EDGE, 16)]
                fidx = dst_v[pl.ds(e * _D_EDGE, 16)] * _D_EDGE + cidx - base_flat
                ok = (fidx >= 0) & (fidx < _HALF * _D_EDGE)
                safe = jnp.where(ok, fidx, _HALF * _D_EDGE + cidx)
                plsc.addupdate_scatter(acc, [safe], row)
                return carry2

            lax.fori_loop(0, _CKM, edge, 0)
            return carry

        lax.fori_loop(0, _NCHM, chunk, 0)
        pltpu.sync_copy(acc.at[pl.ds(0, _HALF * _D_EDGE)], mp_hbm.at[w, p])


def _sc_m_call(msg3, dstrep3, zer):
    mesh = plsc.VectorSubcoreMesh(core_axis_name="c", subcore_axis_name="s")
    f = functools.partial(
        pl.kernel, mesh=mesh,
        compiler_params=pltpu.CompilerParams(needs_layout_passes=False),
        out_type=jax.ShapeDtypeStruct((_NW, 2, _HALF * _D_EDGE), jnp.float32),
        scratch_types=[
            pltpu.VMEM((_HALF * _D_EDGE + 128,), jnp.float32),
            pltpu.VMEM((_CKM * _D_EDGE,), jnp.float32),
            pltpu.VMEM((_CKM * _D_EDGE,), jnp.int32),
        ],
    )(_sc_m_body)
    return f(msg3, dstrep3, zer)


def _henew_body(q_ref, r2_ref, p1t_ref, out_ref):
    out_ref[...] = jnp.maximum(_dot(q_ref[...], p1t_ref[...]) + r2_ref[...], 0.0)


def _henew_call(q, r2, p1t):
    blk = 4000
    grid = _E // blk
    return pl.pallas_call(
        _henew_body,
        grid=(grid,),
        in_specs=[
            pl.BlockSpec((blk, _D_IN), lambda i: (i, 0)),
            pl.BlockSpec((blk, _D_EDGE), lambda i: (i, 0)),
            pl.BlockSpec((_D_IN, _D_EDGE), lambda i: (0, 0)),
        ],
        out_specs=pl.BlockSpec((blk, _D_EDGE), lambda i: (i, 0)),
        out_shape=jax.ShapeDtypeStruct((_E, _D_EDGE), jnp.float32),
    )(q, r2, p1t)


def _msum_body(mp_ref, out_ref):
    out_ref[...] = jnp.sum(mp_ref[...], axis=0)


def _msum_call(mpf):
    blk = 256
    grid = 1280 // blk
    return pl.pallas_call(
        _msum_body,
        grid=(grid,),
        in_specs=[pl.BlockSpec((_NW, blk, 128), lambda i: (0, i, 0))],
        out_specs=pl.BlockSpec((blk, 128), lambda i: (i, 0)),
        out_shape=jax.ShapeDtypeStruct((1280, 128), jnp.float32),
    )(mpf)


def _hnew_body(haa_ref, m_ref, a1t_ref, a2t_ref, ab_ref,
               rowt_ref, rob_ref, hnew_ref, z_ref, mom_ref):
    i = pl.program_id(0)
    m = m_ref[...]
    hn = jnp.maximum(_dot(haa_ref[...], a1t_ref[...]) + _dot(m, a2t_ref[...])
                     + ab_ref[...], 0.0)
    hnew_ref[...] = hn
    z = _dot(hn, rowt_ref[...]) + rob_ref[...]
    z_ref[...] = z

    @pl.when(i == 0)
    def _():
        mom_ref[...] = jnp.zeros_like(mom_ref)

    mom_ref[...] += jnp.concatenate(
        [jnp.sum(z, axis=0, keepdims=True),
         jnp.sum(z * z, axis=0, keepdims=True)], axis=0)


def _hnew_call(haa, m, a1t, a2t, ab, rowt, rob):
    blk = 2000
    grid = _N // blk
    return pl.pallas_call(
        _hnew_body,
        grid=(grid,),
        in_specs=[
            pl.BlockSpec((blk, _D_IN), lambda i: (i, 0)),
            pl.BlockSpec((blk, _D_EDGE), lambda i: (i, 0)),
            pl.BlockSpec((_D_IN, _D_IN), lambda i: (0, 0)),
            pl.BlockSpec((_D_EDGE, _D_IN), lambda i: (0, 0)),
            pl.BlockSpec((1, _D_IN), lambda i: (0, 0)),
            pl.BlockSpec((_D_IN, _D_IN), lambda i: (0, 0)),
            pl.BlockSpec((1, _D_IN), lambda i: (0, 0)),
        ],
        out_specs=[
            pl.BlockSpec((blk, _D_IN), lambda i: (i, 0)),
            pl.BlockSpec((blk, _D_IN), lambda i: (i, 0)),
            pl.BlockSpec((2, _D_IN), lambda i: (0, 0)),
        ],
        out_shape=[
            jax.ShapeDtypeStruct((_N, _D_IN), jnp.float32),
            jax.ShapeDtypeStruct((_N, _D_IN), jnp.float32),
            jax.ShapeDtypeStruct((2, _D_IN), jnp.float32),
        ],
    )(haa, m, a1t, a2t, ab, rowt, rob)


def _pool_body(z_ref, gid_ref, mom_ref, rog_ref, robt_ref, gath_ref, acc_ref):
    i = pl.program_id(0)
    ng = pl.num_programs(0)
    mu = mom_ref[0:1, :] * (1.0 / _N)
    var = mom_ref[1:2, :] * (1.0 / _N) - mu * mu
    zc = z_ref[...] - mu
    rd = jnp.tanh(zc * lax.rsqrt(var + 1e-5) * rog_ref[...] + robt_ref[...])
    blk = rd.shape[0]
    den = jnp.zeros((blk, 1), jnp.float32)
    for mm, ss in _MEMBERS:
        den = den + jnp.sum(jnp.exp(-0.5 * ((rd - mm) / ss) ** 2), axis=1,
                            keepdims=True)
    inv = 1.0 / den
    ids = gid_ref[...].reshape(1, blk)
    oh = jnp.where(lax.broadcasted_iota(jnp.int32, (_NG, blk), 0) == ids,
                   1.0, 0.0)

    @pl.when(i == 0)
    def _():
        acc_ref[...] = jnp.zeros_like(acc_ref)

    for k, (mm, ss) in enumerate(_MEMBERS):
        g = jnp.exp(-0.5 * ((rd - mm) / ss) ** 2) * inv
        acc_ref[:, k * _D_IN:(k + 1) * _D_IN] += _dot(oh, g)

    @pl.when(i == ng - 1)
    def _():
        gath_ref[...] = jnp.tanh(acc_ref[...])


def _pool_call(z, gid3, mom, rog, robt):
    blk = 2000
    grid = _N // blk
    nf = len(_MEMBERS) * _D_IN
    return pl.pallas_call(
        _pool_body,
        grid=(grid,),
        in_specs=[
            pl.BlockSpec((blk, _D_IN), lambda i: (i, 0)),
            pl.BlockSpec((1, 1, blk), lambda i: (i, 0, 0)),
            pl.BlockSpec((2, _D_IN), lambda i: (0, 0)),
            pl.BlockSpec((1, _D_IN), lambda i: (0, 0)),
            pl.BlockSpec((1, _D_IN), lambda i: (0, 0)),
        ],
        out_specs=pl.BlockSpec((_NG, nf), lambda i: (0, 0)),
        out_shape=jax.ShapeDtypeStruct((_NG, nf), jnp.float32),
        scratch_shapes=[pltpu.VMEM((_NG, nf), jnp.float32)],
    )(z, gid3, mom, rog, robt)


def kernel(hv, he, edge_index, graph_ids, AA_w, AA_b, PA_w, PA_b, A_w, A_b,
           AP_w, AP_b, PP_w, PP_b, P_w, P_b, RO_w, RO_b, RO_gamma, RO_beta):
    f32 = jnp.float32
    wt = jnp.concatenate([AP_w[:, :_D_IN].T, AP_w[:, _D_IN:].T], axis=1)
    bt = jnp.concatenate([jnp.zeros((_D_IN,), f32), AP_b]).reshape(1, -1)
    waa = AA_w.T
    baa = AA_b.reshape(1, -1)
    wc = jnp.concatenate([PA_w.T, PP_w.T], axis=1)
    bc = jnp.concatenate([PA_b, PP_b]).reshape(1, -1)
    p2t = P_w[:, _D_IN:].T
    pb = P_b.reshape(1, -1)
    p1t = P_w[:, :_D_IN].T
    a1t = A_w[:, :_D_IN].T
    a2t = A_w[:, _D_IN:].T
    ab = A_b.reshape(1, -1)
    rowt = RO_w.T
    rob = RO_b.reshape(1, -1)
    rog = RO_gamma.reshape(1, -1)
    robt = RO_beta.reshape(1, -1)

    src3 = edge_index[0].reshape(_NW, _NCH, _CK)
    dst3 = edge_index[1].reshape(_NW, _NCH, _CK)
    dstrep3 = jnp.broadcast_to(edge_index[1][:, None],
                               (_E, _D_EDGE)).reshape(_NW, _NCHM, _CKM * _D_EDGE)
    zer = jnp.zeros((_HALF * _D_EDGE,), f32)
    gid3 = graph_ids.reshape(_N // 2000, 1, 2000)

    t_tab, haa = _node_call(hv, wt, bt, waa, baa)
    msg, r2 = _edgepre_call(he, wc, bc, p2t, pb)
    q = _sc_q_call(t_tab, src3, dst3)
    mp = _sc_m_call(msg.reshape(_NW, _NCHM, _CKM * _D_EDGE), dstrep3, zer)
    he_new = _henew_call(q, r2, p1t)
    msum = _msum_call(mp.reshape(_NW, 1280, 128))
    m2d = msum.reshape(_NPAD, _D_EDGE)[:_N]
    h_new, z, mom = _hnew_call(haa, m2d, a1t, a2t, ab, rowt, rob)
    gathered = _pool_call(z, gid3, mom, rog, robt)
    return (h_new, he_new, gathered)


# speedup vs baseline: 1.1611x; 1.1611x over previous
import functools

import jax
import jax.numpy as jnp
from jax import lax
from jax.experimental import pallas as pl
from jax.experimental.pallas import tpu as pltpu
from jax.experimental.pallas import tpu_sc as plsc

_N = 10000
_E = 320000
_D_IN = 128
_D_EDGE = 16
_NG = 64
_MEMBERS = [(-1.645, 0.283), (-1.08, 0.17), (-0.739, 0.134), (-0.468, 0.118),
            (-0.228, 0.114), (0.0, 0.114), (0.228, 0.114), (0.468, 0.118),
            (0.739, 0.134), (1.08, 0.17), (1.645, 0.283)]

_NC = 2
_NS = 16
_NW = _NC * _NS
_EPW = _E // _NW
_CK = 80
_NCH = _EPW // _CK
_NPAD = 10240
_HALF = _NPAD // 2
_CKM = 1000
_NCHM = _EPW // _CKM

_HP = lax.Precision.HIGHEST


def _dot(a, b):
    return jnp.dot(a, b, precision=_HP, preferred_element_type=jnp.float32)


def _node_body(hv_ref, wt_ref, bt_ref, waa_ref, baa_ref, t_ref, haa_ref):
    x = hv_ref[...]
    t_ref[...] = _dot(x, wt_ref[...]) + bt_ref[...]
    haa_ref[...] = jnp.maximum(_dot(x, waa_ref[...]) + baa_ref[...], 0.0)


def _node_call(hv, wt, bt, waa, baa):
    blk = 2000
    grid = _N // blk
    return pl.pallas_call(
        _node_body,
        grid=(grid,),
        in_specs=[
            pl.BlockSpec((blk, _D_IN), lambda i: (i, 0)),
            pl.BlockSpec((_D_IN, 2 * _D_IN), lambda i: (0, 0)),
            pl.BlockSpec((1, 2 * _D_IN), lambda i: (0, 0)),
            pl.BlockSpec((_D_IN, _D_IN), lambda i: (0, 0)),
            pl.BlockSpec((1, _D_IN), lambda i: (0, 0)),
        ],
        out_specs=[
            pl.BlockSpec((blk, 2 * _D_IN), lambda i: (i, 0)),
            pl.BlockSpec((blk, _D_IN), lambda i: (i, 0)),
        ],
        out_shape=[
            jax.ShapeDtypeStruct((_N, 2 * _D_IN), jnp.float32),
            jax.ShapeDtypeStruct((_N, _D_IN), jnp.float32),
        ],
    )(hv, wt, bt, waa, baa)


def _edgepre_body(he_ref, wc_ref, bc_ref, p2t_ref, pb_ref, msg_ref, r2_ref):
    t = jnp.maximum(_dot(he_ref[...], wc_ref[...]) + bc_ref[...], 0.0)
    msg_ref[...] = t[:, :_D_EDGE]
    r2_ref[...] = _dot(t[:, _D_EDGE:], p2t_ref[...]) + pb_ref[...]


def _edgepre_call(he, wc, bc, p2t, pb):
    blk = 8000
    grid = _E // blk
    return pl.pallas_call(
        _edgepre_body,
        grid=(grid,),
        in_specs=[
            pl.BlockSpec((blk, _D_EDGE), lambda i: (i, 0)),
            pl.BlockSpec((_D_EDGE, 2 * _D_EDGE), lambda i: (0, 0)),
            pl.BlockSpec((1, 2 * _D_EDGE), lambda i: (0, 0)),
            pl.BlockSpec((_D_EDGE, _D_EDGE), lambda i: (0, 0)),
            pl.BlockSpec((1, _D_EDGE), lambda i: (0, 0)),
        ],
        out_specs=[
            pl.BlockSpec((blk, _D_EDGE), lambda i: (i, 0)),
            pl.BlockSpec((blk, _D_EDGE), lambda i: (i, 0)),
        ],
        out_shape=[
            jax.ShapeDtypeStruct((_E, _D_EDGE), jnp.float32),
            jax.ShapeDtypeStruct((_E, _D_EDGE), jnp.float32),
        ],
    )(he, wc, bc, p2t, pb)


def _sc_q_body(t_hbm, src2_hbm, dst2_hbm, q_hbm,
               idx_s, idx_d, rows_s, rows_d, out_v,
               sem_s0, sem_s1, sem_d0, sem_d1, sem_o0, sem_o1):
    c = lax.axis_index("c")
    s = lax.axis_index("s")
    w = c * _NS + s
    base = w * _EPW
    sems_s = (sem_s0, sem_s1)
    sems_d = (sem_d0, sem_d1)
    sems_o = (sem_o0, sem_o1)

    pltpu.sync_copy(src2_hbm.at[w], idx_s)
    pltpu.sync_copy(dst2_hbm.at[w], idx_d)

    def gstart(j, b):
        pltpu.async_copy(t_hbm.at[idx_s.at[pl.ds(j * _CK, _CK)]],
                         rows_s.at[pl.ds(b * _CK, _CK)], sems_s[b])
        pltpu.async_copy(t_hbm.at[idx_d.at[pl.ds(j * _CK, _CK)]],
                         rows_d.at[pl.ds(b * _CK, _CK)], sems_d[b])

    def gwait(j, b):
        pltpu.make_async_copy(t_hbm.at[idx_s.at[pl.ds(j * _CK, _CK)]],
                              rows_s.at[pl.ds(b * _CK, _CK)], sems_s[b]).wait()
        pltpu.make_async_copy(t_hbm.at[idx_d.at[pl.ds(j * _CK, _CK)]],
                              rows_d.at[pl.ds(b * _CK, _CK)], sems_d[b]).wait()

    def do_chunk(j, b):
        gwait(j, b)

        @pl.when(j + 1 < _NCH)
        def _():
            gstart(j + 1, 1 - b)

        @pl.when(j >= 2)
        def _():
            pltpu.make_async_copy(out_v.at[pl.ds(b * _CK, _CK)],
                                  q_hbm.at[pl.ds(base + (j - 2) * _CK, _CK)],
                                  sems_o[b]).wait()

        def edge(e, carry2):
            r = b * _CK + e
            for cc in range(_D_IN // 16):
                lo = cc * 16
                a = (rows_s[r, pl.ds(lo, 16)]
                     + rows_d[r, pl.ds(_D_IN + lo, 16)])
                bb = (rows_d[r, pl.ds(lo, 16)]
                      + rows_s[r, pl.ds(_D_IN + lo, 16)])
                out_v[r, pl.ds(lo, 16)] = (jnp.maximum(a, 0.0)
                                           + jnp.maximum(bb, 0.0))
            return carry2

        lax.fori_loop(0, _CK, edge, 0)
        pltpu.async_copy(out_v.at[pl.ds(b * _CK, _CK)],
                         q_hbm.at[pl.ds(base + j * _CK, _CK)], sems_o[b])

    gstart(0, 0)

    def pair(p, carry):
        do_chunk(2 * p, 0)
        do_chunk(2 * p + 1, 1)
        return carry

    lax.fori_loop(0, _NCH // 2, pair, 0)
    do_chunk(_NCH - 1, 0)
    pltpu.make_async_copy(out_v.at[pl.ds(_CK, _CK)],
                          q_hbm.at[pl.ds(base + (_NCH - 2) * _CK, _CK)],
                          sems_o[1]).wait()
    pltpu.make_async_copy(out_v.at[pl.ds(0, _CK)],
                          q_hbm.at[pl.ds(base + (_NCH - 1) * _CK, _CK)],
                          sems_o[0]).wait()


def _sc_q_call(t_tab, src2, dst2):
    mesh = plsc.VectorSubcoreMesh(core_axis_name="c", subcore_axis_name="s")
    f = functools.partial(
        pl.kernel, mesh=mesh,
        out_type=jax.ShapeDtypeStruct((_E, _D_IN), jnp.float32),
        scratch_types=[
            pltpu.VMEM((_EPW,), jnp.int32),
            pltpu.VMEM((_EPW,), jnp.int32),
            pltpu.VMEM((2 * _CK, 2 * _D_IN), jnp.float32),
            pltpu.VMEM((2 * _CK, 2 * _D_IN), jnp.float32),
            pltpu.VMEM((2 * _CK, _D_IN), jnp.float32),
            pltpu.SemaphoreType.DMA,
            pltpu.SemaphoreType.DMA,
            pltpu.SemaphoreType.DMA,
            pltpu.SemaphoreType.DMA,
            pltpu.SemaphoreType.DMA,
            pltpu.SemaphoreType.DMA,
        ],
    )(_sc_q_body)
    return f(t_tab, src2, dst2)


def _sc_m_body(msg3_hbm, dst3_hbm, zer_hbm, mp_hbm,
               acc, msg_v, dst_v):
    c = lax.axis_index("c")
    s = lax.axis_index("s")
    w = c * _NS + s
    cidx = lax.iota(jnp.int32, 16)

    for p in range(2):
        base_flat = p * _HALF * _D_EDGE
        pltpu.sync_copy(zer_hbm, acc.at[pl.ds(0, _HALF * _D_EDGE)])

        def chunk(j, carry):
            pltpu.sync_copy(msg3_hbm.at[w, j], msg_v)
            pltpu.sync_copy(dst3_hbm.at[w, j], dst_v)

            def edge(e4, carry2):
                for u in range(4):
                    e = e4 * 4 + u
                    row = msg_v[pl.ds(e * _D_EDGE, 16)]
                    fidx = (dst_v[pl.ds(e * _D_EDGE, 16)] * _D_EDGE + cidx
                            - base_flat)
                    ok = (fidx >= 0) & (fidx < _HALF * _D_EDGE)
                    safe = jnp.where(ok, fidx, _HALF * _D_EDGE + cidx)
                    plsc.addupdate_scatter(acc, [safe], row)
                return carry2

            lax.fori_loop(0, _CKM // 4, edge, 0)
            return carry

        lax.fori_loop(0, _NCHM, chunk, 0)
        pltpu.sync_copy(acc.at[pl.ds(0, _HALF * _D_EDGE)], mp_hbm.at[w, p])


def _sc_m_call(msg3, dstrep3, zer):
    mesh = plsc.VectorSubcoreMesh(core_axis_name="c", subcore_axis_name="s")
    f = functools.partial(
        pl.kernel, mesh=mesh,
        compiler_params=pltpu.CompilerParams(needs_layout_passes=False),
        out_type=jax.ShapeDtypeStruct((_NW, 2, _HALF * _D_EDGE), jnp.float32),
        scratch_types=[
            pltpu.VMEM((_HALF * _D_EDGE + 128,), jnp.float32),
            pltpu.VMEM((_CKM * _D_EDGE,), jnp.float32),
            pltpu.VMEM((_CKM * _D_EDGE,), jnp.int32),
        ],
    )(_sc_m_body)
    return f(msg3, dstrep3, zer)


def _henew_body(q_ref, r2_ref, p1t_ref, out_ref):
    out_ref[...] = jnp.maximum(_dot(q_ref[...], p1t_ref[...]) + r2_ref[...], 0.0)


def _henew_call(q, r2, p1t):
    blk = 4000
    grid = _E // blk
    return pl.pallas_call(
        _henew_body,
        grid=(grid,),
        in_specs=[
            pl.BlockSpec((blk, _D_IN), lambda i: (i, 0)),
            pl.BlockSpec((blk, _D_EDGE), lambda i: (i, 0)),
            pl.BlockSpec((_D_IN, _D_EDGE), lambda i: (0, 0)),
        ],
        out_specs=pl.BlockSpec((blk, _D_EDGE), lambda i: (i, 0)),
        out_shape=jax.ShapeDtypeStruct((_E, _D_EDGE), jnp.float32),
    )(q, r2, p1t)


def _msum_body(mp_ref, out_ref):
    out_ref[...] = jnp.sum(mp_ref[...], axis=0)


def _msum_call(mpf):
    blk = 256
    grid = 1280 // blk
    return pl.pallas_call(
        _msum_body,
        grid=(grid,),
        in_specs=[pl.BlockSpec((_NW, blk, 128), lambda i: (0, i, 0))],
        out_specs=pl.BlockSpec((blk, 128), lambda i: (i, 0)),
        out_shape=jax.ShapeDtypeStruct((1280, 128), jnp.float32),
    )(mpf)


def _hnew_body(haa_ref, m_ref, a1t_ref, a2t_ref, ab_ref,
               rowt_ref, rob_ref, hnew_ref, z_ref, mom_ref):
    i = pl.program_id(0)
    m = m_ref[...]
    hn = jnp.maximum(_dot(haa_ref[...], a1t_ref[...]) + _dot(m, a2t_ref[...])
                     + ab_ref[...], 0.0)
    hnew_ref[...] = hn
    z = _dot(hn, rowt_ref[...]) + rob_ref[...]
    z_ref[...] = z

    @pl.when(i == 0)
    def _():
        mom_ref[...] = jnp.zeros_like(mom_ref)

    mom_ref[...] += jnp.concatenate(
        [jnp.sum(z, axis=0, keepdims=True),
         jnp.sum(z * z, axis=0, keepdims=True)], axis=0)


def _hnew_call(haa, m, a1t, a2t, ab, rowt, rob):
    blk = 2000
    grid = _N // blk
    return pl.pallas_call(
        _hnew_body,
        grid=(grid,),
        in_specs=[
            pl.BlockSpec((blk, _D_IN), lambda i: (i, 0)),
            pl.BlockSpec((blk, _D_EDGE), lambda i: (i, 0)),
            pl.BlockSpec((_D_IN, _D_IN), lambda i: (0, 0)),
            pl.BlockSpec((_D_EDGE, _D_IN), lambda i: (0, 0)),
            pl.BlockSpec((1, _D_IN), lambda i: (0, 0)),
            pl.BlockSpec((_D_IN, _D_IN), lambda i: (0, 0)),
            pl.BlockSpec((1, _D_IN), lambda i: (0, 0)),
        ],
        out_specs=[
            pl.BlockSpec((blk, _D_IN), lambda i: (i, 0)),
            pl.BlockSpec((blk, _D_IN), lambda i: (i, 0)),
            pl.BlockSpec((2, _D_IN), lambda i: (0, 0)),
        ],
        out_shape=[
            jax.ShapeDtypeStruct((_N, _D_IN), jnp.float32),
            jax.ShapeDtypeStruct((_N, _D_IN), jnp.float32),
            jax.ShapeDtypeStruct((2, _D_IN), jnp.float32),
        ],
    )(haa, m, a1t, a2t, ab, rowt, rob)


def _pool_body(z_ref, gid_ref, mom_ref, rog_ref, robt_ref, gath_ref, acc_ref):
    i = pl.program_id(0)
    ng = pl.num_programs(0)
    mu = mom_ref[0:1, :] * (1.0 / _N)
    var = mom_ref[1:2, :] * (1.0 / _N) - mu * mu
    zc = z_ref[...] - mu
    rd = jnp.tanh(zc * lax.rsqrt(var + 1e-5) * rog_ref[...] + robt_ref[...])
    blk = rd.shape[0]
    den = jnp.zeros((blk, 1), jnp.float32)
    for mm, ss in _MEMBERS:
        den = den + jnp.sum(jnp.exp(-0.5 * ((rd - mm) / ss) ** 2), axis=1,
                            keepdims=True)
    inv = 1.0 / den
    ids = gid_ref[...].reshape(1, blk)
    oh = jnp.where(lax.broadcasted_iota(jnp.int32, (_NG, blk), 0) == ids,
                   1.0, 0.0)

    @pl.when(i == 0)
    def _():
        acc_ref[...] = jnp.zeros_like(acc_ref)

    for k, (mm, ss) in enumerate(_MEMBERS):
        g = jnp.exp(-0.5 * ((rd - mm) / ss) ** 2) * inv
        acc_ref[:, k * _D_IN:(k + 1) * _D_IN] += _dot(oh, g)

    @pl.when(i == ng - 1)
    def _():
        gath_ref[...] = jnp.tanh(acc_ref[...])


def _pool_call(z, gid3, mom, rog, robt):
    blk = 2000
    grid = _N // blk
    nf = len(_MEMBERS) * _D_IN
    return pl.pallas_call(
        _pool_body,
        grid=(grid,),
        in_specs=[
            pl.BlockSpec((blk, _D_IN), lambda i: (i, 0)),
            pl.BlockSpec((1, 1, blk), lambda i: (i, 0, 0)),
            pl.BlockSpec((2, _D_IN), lambda i: (0, 0)),
            pl.BlockSpec((1, _D_IN), lambda i: (0, 0)),
            pl.BlockSpec((1, _D_IN), lambda i: (0, 0)),
        ],
        out_specs=pl.BlockSpec((_NG, nf), lambda i: (0, 0)),
        out_shape=jax.ShapeDtypeStruct((_NG, nf), jnp.float32),
        scratch_shapes=[pltpu.VMEM((_NG, nf), jnp.float32)],
    )(z, gid3, mom, rog, robt)


def kernel(hv, he, edge_index, graph_ids, AA_w, AA_b, PA_w, PA_b, A_w, A_b,
           AP_w, AP_b, PP_w, PP_b, P_w, P_b, RO_w, RO_b, RO_gamma, RO_beta):
    f32 = jnp.float32
    wt = jnp.concatenate([AP_w[:, :_D_IN].T, AP_w[:, _D_IN:].T], axis=1)
    bt = jnp.concatenate([jnp.zeros((_D_IN,), f32), AP_b]).reshape(1, -1)
    waa = AA_w.T
    baa = AA_b.reshape(1, -1)
    wc = jnp.concatenate([PA_w.T, PP_w.T], axis=1)
    bc = jnp.concatenate([PA_b, PP_b]).reshape(1, -1)
    p2t = P_w[:, _D_IN:].T
    pb = P_b.reshape(1, -1)
    p1t = P_w[:, :_D_IN].T
    a1t = A_w[:, :_D_IN].T
    a2t = A_w[:, _D_IN:].T
    ab = A_b.reshape(1, -1)
    rowt = RO_w.T
    rob = RO_b.reshape(1, -1)
    rog = RO_gamma.reshape(1, -1)
    robt = RO_beta.reshape(1, -1)

    src2 = edge_index[0].reshape(_NW, _EPW)
    dst2 = edge_index[1].reshape(_NW, _EPW)
    dstrep3 = jnp.broadcast_to(edge_index[1][:, None],
                               (_E, _D_EDGE)).reshape(_NW, _NCHM, _CKM * _D_EDGE)
    zer = jnp.zeros((_HALF * _D_EDGE,), f32)
    gid3 = graph_ids.reshape(_N // 2000, 1, 2000)

    t_tab, haa = _node_call(hv, wt, bt, waa, baa)
    msg, r2 = _edgepre_call(he, wc, bc, p2t, pb)
    q = _sc_q_call(t_tab, src2, dst2)
    mp = _sc_m_call(msg.reshape(_NW, _NCHM, _CKM * _D_EDGE), dstrep3, zer)
    he_new = _henew_call(q, r2, p1t)
    msum = _msum_call(mp.reshape(_NW, 1280, 128))
    m2d = msum.reshape(_NPAD, _D_EDGE)[:_N]
    h_new, z, mom = _hnew_call(haa, m2d, a1t, a2t, ab, rowt, rob)
    gathered = _pool_call(z, gid3, mom, rog, robt)
    return (h_new, he_new, gathered)
